# Initial kernel scaffold; baseline (speedup 1.0000x reference)
import jax
import jax.numpy as jnp
from jax.experimental import pallas as pl


def kernel(x, edge_index, edge_weight, edge_attr, batch, p1, W1, b1, Wih1, Whh1, bih1, bhh1, W2, b2, Wih2, Whh2, bih2, bhh2):
    raise NotImplementedError("write your pallas kernel here")


# speedup vs baseline: 10.4499x; 10.4499x over previous
import functools

import jax
import jax.numpy as jnp
from jax import lax
from jax.experimental import pallas as pl
from jax.experimental.pallas import tpu as pltpu
from jax.experimental.pallas import tpu_sc as plsc

N = 10000
E = 320000
D = 128
K = 128

NC = 2
NS = 16
NW = NC * NS
L = 16

C = 128
EPW = 10368
E_PAD = EPW * NW
NCH = EPW // C

A3 = 80
B3 = 125

_sc_mesh = functools.partial(
    plsc.VectorSubcoreMesh, core_axis_name="c", subcore_axis_name="s")
_sc_params = pltpu.CompilerParams(needs_layout_passes=False)


NDEG = 10240


@functools.partial(
    pl.kernel,
    out_type=jax.ShapeDtypeStruct((NC, NDEG), jnp.float32),
    mesh=_sc_mesh(),
    compiler_params=_sc_params,
    scratch_types=[
        pltpu.VMEM((C,), jnp.int32),
        pltpu.VMEM((C,), jnp.float32),
        pltpu.VMEM((640,), jnp.float32),
        pltpu.VMEM_SHARED((NDEG,), jnp.float32),
    ],
)
def _deg_kernel(dst_hbm, ew_hbm, out_hbm, idx_v, ew_v, stage_v, deg_sh):
    cid = lax.axis_index("c")
    sid = lax.axis_index("s")
    wid = sid * NC + cid

    def zstage(i, _):
        stage_v[pl.ds(i * L, L)] = jnp.zeros((L,), jnp.float32)
        return 0

    lax.fori_loop(0, 640 // L, zstage, 0)
    pltpu.sync_copy(stage_v, deg_sh.at[pl.ds(sid * 640, 640)])
    plsc.subcore_barrier()

    def chunk(i, _):
        base = wid * EPW + i * C
        pltpu.sync_copy(dst_hbm.at[pl.ds(base, C)], idx_v)
        pltpu.sync_copy(ew_hbm.at[pl.ds(base, C)], ew_v)
        pltpu.sync_copy(ew_v, deg_sh.at[idx_v], add=True)
        return 0

    lax.fori_loop(0, NCH, chunk, 0)
    plsc.subcore_barrier()
    pltpu.sync_copy(deg_sh.at[pl.ds(sid * 640, 640)], stage_v)
    pltpu.sync_copy(stage_v, out_hbm.at[cid, pl.ds(sid * 640, 640)])


NACC = 10240
RPT = NACC // NS


@functools.partial(
    pl.kernel,
    out_type=jax.ShapeDtypeStruct((NC, NACC, D), jnp.float32),
    mesh=_sc_mesh(),
    compiler_params=_sc_params,
    scratch_types=[
        pltpu.VMEM((NDEG,), jnp.float32),
        pltpu.VMEM((C,), jnp.int32),
        pltpu.VMEM((C,), jnp.int32),
        pltpu.VMEM((C,), jnp.float32),
        pltpu.VMEM((C, D), jnp.float32),
        pltpu.VMEM_SHARED((NACC, D), jnp.float32),
        pltpu.SemaphoreType.DMA,
    ],
)
def _gcn_kernel(h_hbm, dinv_hbm, src_hbm, dst_hbm, ew_hbm, out_hbm,
                dinv_v, src_v, dst_v, ew_v, rows_v, acc_sh, sem):
    cid = lax.axis_index("c")
    sid = lax.axis_index("s")
    wid = sid * NC + cid

    def zrow(i, _):
        rows_v[i // 8, pl.ds((i % 8) * L, L)] = jnp.zeros((L,), jnp.float32)
        return 0

    lax.fori_loop(0, C * 8, zrow, 0)
    for q in range(RPT // C):
        pltpu.sync_copy(rows_v, acc_sh.at[pl.ds(sid * RPT + q * C, C)])
    pltpu.sync_copy(dinv_hbm, dinv_v)
    plsc.subcore_barrier()

    def chunk(i, _):
        base = wid * EPW + i * C
        pltpu.sync_copy(src_hbm.at[pl.ds(base, C)], src_v)
        pltpu.sync_copy(dst_hbm.at[pl.ds(base, C)], dst_v)
        pltpu.sync_copy(ew_hbm.at[pl.ds(base, C)], ew_v)
        pltpu.async_copy(h_hbm.at[src_v], rows_v, sem).wait()

        def group(g, _):
            s16 = src_v[pl.ds(g * L, L)]
            d16 = dst_v[pl.ds(g * L, L)]
            w16 = ew_v[pl.ds(g * L, L)]
            coef = (w16 * plsc.load_gather(dinv_v, [s16])
                    * plsc.load_gather(dinv_v, [d16]))
            for j in range(L):
                e = g * L + j
                cj = jnp.full((L,), coef[j], jnp.float32)
                for q in range(D // L):
                    sl = rows_v[e, pl.ds(q * L, L)]
                    rows_v[e, pl.ds(q * L, L)] = sl * cj
            return 0

        lax.fori_loop(0, C // L, group, 0)
        pltpu.sync_copy(rows_v, acc_sh.at[dst_v], add=True)
        return 0

    lax.fori_loop(0, NCH, chunk, 0)
    plsc.subcore_barrier()

    for q in range(RPT // C):
        pltpu.sync_copy(acc_sh.at[pl.ds(sid * RPT + q * C, C)], rows_v)
        pltpu.sync_copy(rows_v, out_hbm.at[cid, pl.ds(sid * RPT + q * C, C)])


def _gru_new_weight(Xt, H, WihT, WhhT, bih, bhh):
    gi = jax.lax.dot(Xt, WihT) + bih[None, :]
    gh = jax.lax.dot(H, WhhT) + bhh[None, :]
    r = jax.nn.sigmoid(gi[:, :D] + gh[:, :D])
    z = jax.nn.sigmoid(gi[:, D:2 * D] + gh[:, D:2 * D])
    n = jnp.tanh(gi[:, 2 * D:] + r * gh[:, 2 * D:])
    return (1.0 - z) * n + z * H


def _topk_gru_body(x3_ref, sc_ref, th_ref, w_ref, wiht_ref, whht_ref,
                   bih_ref, bhh_ref, wn_ref, score_ref, xt_ref):
    score_ref[...] = sc_ref[...]
    ia = lax.broadcasted_iota(jnp.int32, (A3, B3), 0)
    ib = lax.broadcasted_iota(jnp.int32, (A3, B3), 1)
    flat_iota = ia * B3 + ib

    def step(k, _):
        s = score_ref[...]
        m = jnp.max(s)
        flat = jnp.min(jnp.where(s == m, flat_iota, jnp.int32(2 ** 30)))
        a = flat // B3
        b = flat - a * B3
        row = x3_ref[pl.ds(a, 1), pl.ds(b, 1), :]
        tv = jnp.sum(jnp.where(flat_iota == flat, th_ref[...], 0.0))
        xt_ref[pl.ds(k, 1), :] = jnp.reshape(row, (1, D)) * tv
        score_ref[...] = jnp.where(flat_iota == flat, -jnp.inf, s)
        return 0

    lax.fori_loop(0, K, step, 0)
    wn_ref[...] = _gru_new_weight(xt_ref[...], w_ref[...], wiht_ref[...],
                                  whht_ref[...], bih_ref[...], bhh_ref[...])


def _topk_gru(x3, score2d, tanh2d, W, WihT, WhhT, bih, bhh):
    return pl.pallas_call(
        _topk_gru_body,
        out_shape=jax.ShapeDtypeStruct((D, D), jnp.float32),
        scratch_shapes=[
            pltpu.VMEM((A3, B3), jnp.float32),
            pltpu.VMEM((K, D), jnp.float32),
        ],
    )(x3, score2d, tanh2d, W, WihT, WhhT, bih, bhh)


def _dinv_body(deg_ref, out_ref):
    d = deg_ref[0, :] + deg_ref[1, :]
    out_ref[...] = jax.lax.rsqrt(d)


def _dinv(deg2):
    return pl.pallas_call(
        _dinv_body,
        out_shape=jax.ShapeDtypeStruct((NDEG,), jnp.float32),
    )(deg2)


_BM = 400


def _mm_body(x_ref, w_ref, o_ref):
    o_ref[...] = jax.lax.dot(x_ref[...], w_ref[...])


def _mm(x, w):
    return pl.pallas_call(
        _mm_body,
        grid=(N // _BM,),
        in_specs=[
            pl.BlockSpec((_BM, D), lambda i: (i, 0)),
            pl.BlockSpec((D, D), lambda i: (0, 0)),
        ],
        out_specs=pl.BlockSpec((_BM, D), lambda i: (i, 0)),
        out_shape=jax.ShapeDtypeStruct((N, D), jnp.float32),
    )(x, w)


def _combine_body(s0_ref, s1_ref, b_ref, o_ref):
    o_ref[...] = s0_ref[0] + s1_ref[0] + b_ref[...]


def _combine(S, b):
    return pl.pallas_call(
        _combine_body,
        grid=(N // _BM,),
        in_specs=[
            pl.BlockSpec((1, _BM, D), lambda i: (0, i, 0)),
            pl.BlockSpec((1, _BM, D), lambda i: (1, i, 0)),
            pl.BlockSpec((1, D), lambda i: (0, 0)),
        ],
        out_specs=pl.BlockSpec((_BM, D), lambda i: (i, 0)),
        out_shape=jax.ShapeDtypeStruct((N, D), jnp.float32),
    )(S, S, b.reshape(1, D))


def kernel(x, edge_index, edge_weight, edge_attr, batch, p1, W1, b1, Wih1,
           Whh1, bih1, bhh1, W2, b2, Wih2, Whh2, bih2, bhh2):
    del edge_attr, batch
    ei = edge_index.astype(jnp.int32)
    pad = E_PAD - E - N
    loop = jnp.arange(N, dtype=jnp.int32)
    padi = jnp.arange(pad, dtype=jnp.int32)
    src = jnp.concatenate([ei[0], loop, padi])
    dst = jnp.concatenate([ei[1], loop, padi])
    ew = jnp.concatenate([
        edge_weight.astype(jnp.float32),
        jnp.ones((N,), jnp.float32),
        jnp.zeros((pad,), jnp.float32),
    ])
    x3 = x.reshape(A3, B3, D)

    deg2 = _deg_kernel(dst, ew)
    dinv = _dinv(deg2)

    pnorm = jnp.linalg.norm(p1) + 1e-12
    score1 = (x @ p1 / pnorm).reshape(A3, B3)
    W1n = _topk_gru(x3, score1, jnp.tanh(score1), W1, Wih1.T, Whh1.T,
                    bih1, bhh1)
    h1 = _mm(x, W1n)
    S1 = _gcn_kernel(h1, dinv, src, dst, ew)
    out1 = _combine(S1, b1)

    score2 = (out1 @ p1 / pnorm).reshape(A3, B3)
    W2n = _topk_gru(out1.reshape(A3, B3, D), score2, jnp.tanh(score2),
                    W2, Wih2.T, Whh2.T, bih2, bhh2)
    h2 = _mm(out1, W2n)
    S2 = _gcn_kernel(h2, dinv, src, dst, ew)
    out2 = _combine(S2, b2)
    return out2[None]


# speedup vs baseline: 13.2529x; 1.2682x over previous
import functools

import jax
import jax.numpy as jnp
from jax import lax
from jax.experimental import pallas as pl
from jax.experimental.pallas import tpu as pltpu
from jax.experimental.pallas import tpu_sc as plsc

N = 10000
E = 320000
D = 128
K = 128

NC = 2
NS = 16
NW = NC * NS
L = 16

C = 128
EPW = 10496
E_PAD = EPW * NW
NCH = EPW // C

A3 = 80
B3 = 125

_sc_mesh = functools.partial(
    plsc.VectorSubcoreMesh, core_axis_name="c", subcore_axis_name="s")
_sc_params = pltpu.CompilerParams(needs_layout_passes=False)


NDEG = 10240


@functools.partial(
    pl.kernel,
    out_type=jax.ShapeDtypeStruct((NC, NDEG), jnp.float32),
    mesh=_sc_mesh(),
    compiler_params=_sc_params,
    scratch_types=[
        pltpu.VMEM((C,), jnp.int32),
        pltpu.VMEM((C,), jnp.float32),
        pltpu.VMEM((640,), jnp.float32),
        pltpu.VMEM_SHARED((NDEG,), jnp.float32),
    ],
)
def _deg_kernel(dst_hbm, ew_hbm, out_hbm, idx_v, ew_v, stage_v, deg_sh):
    cid = lax.axis_index("c")
    sid = lax.axis_index("s")
    wid = sid * NC + cid

    def zstage(i, _):
        stage_v[pl.ds(i * L, L)] = jnp.zeros((L,), jnp.float32)
        return 0

    lax.fori_loop(0, 640 // L, zstage, 0)
    pltpu.sync_copy(stage_v, deg_sh.at[pl.ds(sid * 640, 640)])
    plsc.subcore_barrier()

    def chunk(i, _):
        base = wid * EPW + i * C
        pltpu.sync_copy(dst_hbm.at[pl.ds(base, C)], idx_v)
        pltpu.sync_copy(ew_hbm.at[pl.ds(base, C)], ew_v)
        pltpu.sync_copy(ew_v, deg_sh.at[idx_v], add=True)
        return 0

    lax.fori_loop(0, NCH, chunk, 0)
    plsc.subcore_barrier()
    pltpu.sync_copy(deg_sh.at[pl.ds(sid * 640, 640)], stage_v)
    pltpu.sync_copy(stage_v, out_hbm.at[cid, pl.ds(sid * 640, 640)])


NACC = 10240
RPT = NACC // NS


@functools.partial(
    pl.kernel,
    out_type=jax.ShapeDtypeStruct((NC, NACC, D), jnp.float32),
    mesh=_sc_mesh(),
    compiler_params=_sc_params,
    scratch_types=[
        pltpu.VMEM((NDEG,), jnp.float32),
        pltpu.VMEM((C,), jnp.int32),
        pltpu.VMEM((C,), jnp.int32),
        pltpu.VMEM((C,), jnp.float32),
        pltpu.VMEM((C, D), jnp.float32),
        pltpu.VMEM((C,), jnp.int32),
        pltpu.VMEM((C,), jnp.int32),
        pltpu.VMEM((C,), jnp.float32),
        pltpu.VMEM((C, D), jnp.float32),
        pltpu.VMEM_SHARED((NACC, D), jnp.float32),
        pltpu.SemaphoreType.DMA,
        pltpu.SemaphoreType.DMA,
    ],
)
def _gcn_kernel(h_hbm, dinv_hbm, src_hbm, dst_hbm, ew_hbm, out_hbm,
                dinv_v, src_v, dst_v, ew_v, rows_v, src_w, dst_w, ew_w,
                rows_w, acc_sh, sem, sem2):
    cid = lax.axis_index("c")
    sid = lax.axis_index("s")
    wid = sid * NC + cid

    def zrow(i, _):
        rows_v[i // 8, pl.ds((i % 8) * L, L)] = jnp.zeros((L,), jnp.float32)
        return 0

    lax.fori_loop(0, C * 8, zrow, 0)
    for q in range(RPT // C):
        pltpu.sync_copy(rows_v, acc_sh.at[pl.ds(sid * RPT + q * C, C)])
    pltpu.sync_copy(dinv_hbm, dinv_v)
    plsc.subcore_barrier()

    def _load_idx(i, sv, dv, wv):
        base = wid * EPW + i * C
        pltpu.sync_copy(src_hbm.at[pl.ds(base, C)], sv)
        pltpu.sync_copy(dst_hbm.at[pl.ds(base, C)], dv)
        pltpu.sync_copy(ew_hbm.at[pl.ds(base, C)], wv)

    def _scale_scatter(sv, dv, wv, rv):
        def group(g, _):
            s16 = sv[pl.ds(g * L, L)]
            d16 = dv[pl.ds(g * L, L)]
            w16 = wv[pl.ds(g * L, L)]
            coef = (w16 * plsc.load_gather(dinv_v, [s16])
                    * plsc.load_gather(dinv_v, [d16]))
            for j in range(L):
                cj = jnp.full((L,), coef[j], jnp.float32)
                for q in range(D // L):
                    sl = rv[g * L + j, pl.ds(q * L, L)]
                    rv[g * L + j, pl.ds(q * L, L)] = sl * cj
            return 0

        lax.fori_loop(0, C // L, group, 0)
        pltpu.sync_copy(rv, acc_sh.at[dv], add=True)

    _load_idx(0, src_v, dst_v, ew_v)
    gA = pltpu.async_copy(h_hbm.at[src_v], rows_v, sem)
    _load_idx(1, src_w, dst_w, ew_w)
    gB = pltpu.async_copy(h_hbm.at[src_w], rows_w, sem2)

    def pair(g, _):
        i = g * 2
        gA_l = pltpu.make_async_copy(h_hbm.at[src_v], rows_v, sem)
        gA_l.wait()
        _scale_scatter(src_v, dst_v, ew_v, rows_v)

        @pl.when(i + 2 < NCH)
        def _():
            _load_idx(i + 2, src_v, dst_v, ew_v)
            pltpu.async_copy(h_hbm.at[src_v], rows_v, sem)

        gB_l = pltpu.make_async_copy(h_hbm.at[src_w], rows_w, sem2)
        gB_l.wait()
        _scale_scatter(src_w, dst_w, ew_w, rows_w)

        @pl.when(i + 3 < NCH)
        def _():
            _load_idx(i + 3, src_w, dst_w, ew_w)
            pltpu.async_copy(h_hbm.at[src_w], rows_w, sem2)

        return 0

    lax.fori_loop(0, NCH // 2, pair, 0)
    plsc.subcore_barrier()

    for q in range(RPT // C):
        pltpu.sync_copy(acc_sh.at[pl.ds(sid * RPT + q * C, C)], rows_v)
        pltpu.sync_copy(rows_v, out_hbm.at[cid, pl.ds(sid * RPT + q * C, C)])


def _gru_new_weight(Xt, H, WihT, WhhT, bih, bhh):
    gi = jax.lax.dot(Xt, WihT) + bih[None, :]
    gh = jax.lax.dot(H, WhhT) + bhh[None, :]
    r = jax.nn.sigmoid(gi[:, :D] + gh[:, :D])
    z = jax.nn.sigmoid(gi[:, D:2 * D] + gh[:, D:2 * D])
    n = jnp.tanh(gi[:, 2 * D:] + r * gh[:, 2 * D:])
    return (1.0 - z) * n + z * H


def _topk_gru_body(x3_ref, sc_ref, th_ref, w_ref, wiht_ref, whht_ref,
                   bih_ref, bhh_ref, wn_ref, score_ref, xt_ref):
    score_ref[...] = sc_ref[...]
    ia = lax.broadcasted_iota(jnp.int32, (A3, B3), 0)
    ib = lax.broadcasted_iota(jnp.int32, (A3, B3), 1)
    flat_iota = ia * B3 + ib

    def step(k, _):
        s = score_ref[...]
        m = jnp.max(s)
        flat = jnp.min(jnp.where(s == m, flat_iota, jnp.int32(2 ** 30)))
        a = flat // B3
        b = flat - a * B3
        row = x3_ref[pl.ds(a, 1), pl.ds(b, 1), :]
        tv = jnp.sum(jnp.where(flat_iota == flat, th_ref[...], 0.0))
        xt_ref[pl.ds(k, 1), :] = jnp.reshape(row, (1, D)) * tv
        score_ref[...] = jnp.where(flat_iota == flat, -jnp.inf, s)
        return 0

    lax.fori_loop(0, K, step, 0)
    wn_ref[...] = _gru_new_weight(xt_ref[...], w_ref[...], wiht_ref[...],
                                  whht_ref[...], bih_ref[...], bhh_ref[...])


def _topk_gru(x3, score2d, tanh2d, W, WihT, WhhT, bih, bhh):
    return pl.pallas_call(
        _topk_gru_body,
        out_shape=jax.ShapeDtypeStruct((D, D), jnp.float32),
        scratch_shapes=[
            pltpu.VMEM((A3, B3), jnp.float32),
            pltpu.VMEM((K, D), jnp.float32),
        ],
    )(x3, score2d, tanh2d, W, WihT, WhhT, bih, bhh)


def _dinv_body(deg_ref, out_ref):
    d = deg_ref[0, :] + deg_ref[1, :]
    out_ref[...] = jax.lax.rsqrt(d)


def _dinv(deg2):
    return pl.pallas_call(
        _dinv_body,
        out_shape=jax.ShapeDtypeStruct((NDEG,), jnp.float32),
    )(deg2)


_BM = 400


def _mm_body(x_ref, w_ref, o_ref):
    o_ref[...] = jax.lax.dot(x_ref[...], w_ref[...])


def _mm(x, w):
    return pl.pallas_call(
        _mm_body,
        grid=(N // _BM,),
        in_specs=[
            pl.BlockSpec((_BM, D), lambda i: (i, 0)),
            pl.BlockSpec((D, D), lambda i: (0, 0)),
        ],
        out_specs=pl.BlockSpec((_BM, D), lambda i: (i, 0)),
        out_shape=jax.ShapeDtypeStruct((N, D), jnp.float32),
    )(x, w)


def _combine_body(s0_ref, s1_ref, b_ref, o_ref):
    o_ref[...] = s0_ref[0] + s1_ref[0] + b_ref[...]


def _combine(S, b):
    return pl.pallas_call(
        _combine_body,
        grid=(N // _BM,),
        in_specs=[
            pl.BlockSpec((1, _BM, D), lambda i: (0, i, 0)),
            pl.BlockSpec((1, _BM, D), lambda i: (1, i, 0)),
            pl.BlockSpec((1, D), lambda i: (0, 0)),
        ],
        out_specs=pl.BlockSpec((_BM, D), lambda i: (i, 0)),
        out_shape=jax.ShapeDtypeStruct((N, D), jnp.float32),
    )(S, S, b.reshape(1, D))


def kernel(x, edge_index, edge_weight, edge_attr, batch, p1, W1, b1, Wih1,
           Whh1, bih1, bhh1, W2, b2, Wih2, Whh2, bih2, bhh2):
    del edge_attr, batch
    ei = edge_index.astype(jnp.int32)
    pad = E_PAD - E - N
    loop = jnp.arange(N, dtype=jnp.int32)
    padi = jnp.arange(pad, dtype=jnp.int32)
    src = jnp.concatenate([ei[0], loop, padi])
    dst = jnp.concatenate([ei[1], loop, padi])
    ew = jnp.concatenate([
        edge_weight.astype(jnp.float32),
        jnp.ones((N,), jnp.float32),
        jnp.zeros((pad,), jnp.float32),
    ])
    x3 = x.reshape(A3, B3, D)

    deg2 = _deg_kernel(dst, ew)
    dinv = _dinv(deg2)

    pnorm = jnp.linalg.norm(p1) + 1e-12
    score1 = (x @ p1 / pnorm).reshape(A3, B3)
    W1n = _topk_gru(x3, score1, jnp.tanh(score1), W1, Wih1.T, Whh1.T,
                    bih1, bhh1)
    h1 = _mm(x, W1n)
    S1 = _gcn_kernel(h1, dinv, src, dst, ew)
    out1 = _combine(S1, b1)

    score2 = (out1 @ p1 / pnorm).reshape(A3, B3)
    W2n = _topk_gru(out1.reshape(A3, B3, D), score2, jnp.tanh(score2),
                    W2, Wih2.T, Whh2.T, bih2, bhh2)
    h2 = _mm(out1, W2n)
    S2 = _gcn_kernel(h2, dinv, src, dst, ew)
    out2 = _combine(S2, b2)
    return out2[None]
